# Initial kernel scaffold; baseline (speedup 1.0000x reference)
import jax
import jax.numpy as jnp
from jax.experimental import pallas as pl


def kernel(x_nchw, w1, b1, w2, b2, g1, gb1, g2, gb2):
    raise NotImplementedError("write your pallas kernel here")


# speedup vs baseline: 1.1126x; 1.1126x over previous
import jax
import jax.numpy as jnp
from jax.experimental import pallas as pl
from jax.experimental.pallas import tpu as pltpu


def _ms_cam_kernel(x_ref, w1_ref, b1_ref, w2_ref, b2_ref,
                   g1_ref, gb1_ref, g2_ref, gb2_ref, o_ref):
    x = x_ref[...]
    C, HW = x.shape

    m = jnp.sum(x, axis=1, keepdims=True) * (1.0 / HW)
    mb = jnp.broadcast_to(m, (C, 128))
    hg = jnp.maximum(
        jnp.dot(g1_ref[...], mb, preferred_element_type=jnp.float32)
        + gb1_ref[...], 0.0)
    xg = (jnp.dot(g2_ref[...], hg, preferred_element_type=jnp.float32)
          + gb2_ref[...])[:, 0:1]

    xb = x.astype(jnp.bfloat16)
    h = jnp.maximum(
        jnp.dot(w1_ref[...], xb, preferred_element_type=jnp.float32)
        + b1_ref[...], 0.0)
    xl = (jnp.dot(w2_ref[...], h.astype(jnp.bfloat16),
                  preferred_element_type=jnp.float32)
          + b2_ref[...])

    o_ref[...] = (x * jax.nn.sigmoid(xl + xg)).astype(o_ref.dtype)


def kernel(x_nchw, w1, b1, w2, b2, g1, gb1, g2, gb2):
    N, C, H, W = x_nchw.shape
    HW = H * W
    Ci = w1.shape[1]

    x = x_nchw.reshape(N, C, HW)

    w1t = w1.T.astype(jnp.bfloat16)
    b1c = b1.reshape(Ci, 1)
    w2t = w2.T.astype(jnp.bfloat16)
    b2c = b2.reshape(C, 1)
    g1t = g1.T
    gb1c = gb1.reshape(Ci, 1)
    g2t = g2.T
    gb2c = gb2.reshape(C, 1)

    const = lambda shape: pl.BlockSpec(shape, lambda n: (0,) * len(shape))
    out = pl.pallas_call(
        _ms_cam_kernel,
        out_shape=jax.ShapeDtypeStruct((N, C, HW), x.dtype),
        grid=(N,),
        in_specs=[
            pl.BlockSpec((None, C, HW), lambda n: (n, 0, 0)),
            const((Ci, C)), const((Ci, 1)),
            const((C, Ci)), const((C, 1)),
            const((Ci, C)), const((Ci, 1)),
            const((C, Ci)), const((C, 1)),
        ],
        out_specs=pl.BlockSpec((None, C, HW), lambda n: (n, 0, 0)),
        compiler_params=pltpu.CompilerParams(
            dimension_semantics=("parallel",)),
    )(x, w1t, b1c, w2t, b2c, g1t, gb1c, g2t, gb2c)

    return out.reshape(N, C, H, W)
